# Initial kernel scaffold; baseline (speedup 1.0000x reference)
import jax
import jax.numpy as jnp
from jax.experimental import pallas as pl


def kernel(x, edge_index, W_gat, att_src, att_dst, gat_bias, W1, b1, W2, b2):
    raise NotImplementedError("write your pallas kernel here")


# speedup vs baseline: 15.7477x; 15.7477x over previous
import dataclasses

import jax
import jax.numpy as jnp
from jax import lax
from jax.experimental import pallas as pl
from jax.experimental.pallas import tpu as pltpu
from jax.experimental.pallas import tpu_sc as plsc

N = 10000
E = 320000
F_IN = 128
HEADS = 8
C = 32
HID = 256
NUM_CLASSES = 6
HALF = 128
LANES = 16
NSC = 2
NTILE = 16
NPAD = 10112
CHUNK = 64
ETOT = E + N
EPAD = ((ETOT + NTILE * CHUNK - 1) // (NTILE * CHUNK)) * (NTILE * CHUNK)
EPT = EPAD // NTILE
NCHUNK = EPT // CHUNK
EPT2 = EPAD // (NSC * NTILE)
NCHUNK2 = EPT2 // CHUNK
NR = 32
NROUND = NPAD // NR
NQ = (NROUND + NTILE - 1) // NTILE


def _prep_body(x_ref, wg_ref, asrc_ref, adst_ref, m_ref, h3_ref, ab_ref):
    h = jnp.dot(x_ref[...], wg_ref[...], preferred_element_type=jnp.float32)
    h3_ref[0:N, :] = h[:, :HALF]
    h3_ref[NPAD:NPAD + N, :] = h[:, HALF:]
    zrows = jnp.zeros((NPAD - N, HALF), jnp.float32)
    h3_ref[N:NPAD, :] = zrows
    h3_ref[NPAD + N:, :] = zrows
    ab_ref[:N, 0:LANES] = jnp.dot(h * asrc_ref[...], m_ref[...],
                                  preferred_element_type=jnp.float32)
    ab_ref[:N, LANES:2 * LANES] = jnp.dot(h * adst_ref[...], m_ref[...],
                                          preferred_element_type=jnp.float32)
    ab_ref[:N, 2 * LANES:] = jnp.zeros((N, HALF - 2 * LANES), jnp.float32)
    ab_ref[N:, :] = jnp.zeros((NPAD - N, HALF), jnp.float32)


def _s1_body(src_hbm, dst_hbm, ab_hbm, h3f_hbm, riota_hbm, outp_hbm, ea_hbm,
             out_sh, isrc, idst, sb, db, eb, hb, zbuf, ridx):
    c = lax.axis_index("c")
    s = lax.axis_index("s")

    @pl.loop(0, NR)
    def _zb(i):
        for j in range(HALF // LANES):
            zbuf[i, pl.ds(LANES * j, LANES)] = jnp.zeros((LANES,), jnp.float32)

    @pl.loop(0, NQ)
    def _init(q):
        cid = s + NTILE * q

        @pl.when(cid < NROUND)
        def _():
            pltpu.sync_copy(riota_hbm.at[pl.ds(cid * NR, NR)], ridx)
            pltpu.sync_copy(zbuf, out_sh.at[ridx])

    plsc.subcore_barrier()

    @pl.loop(0, NCHUNK)
    def _chunk(k):
        base = s * EPT + k * CHUNK
        pltpu.sync_copy(src_hbm.at[pl.ds(base, CHUNK)], isrc)
        pltpu.sync_copy(dst_hbm.at[pl.ds(base, CHUNK)], idst)
        pltpu.sync_copy(ab_hbm.at[isrc], sb)
        pltpu.sync_copy(ab_hbm.at[idst], db)

        @pl.loop(0, CHUNK // LANES)
        def _bias(q):
            isrc[pl.ds(q * LANES, LANES)] = (
                isrc[pl.ds(q * LANES, LANES)] + c * NPAD)

        pltpu.sync_copy(h3f_hbm.at[isrc], hb)

        @pl.loop(0, CHUNK)
        def _edge(e):
            v = sb[e, 0:LANES] + db[e, LANES:2 * LANES]
            v = jnp.maximum(v, 0.2 * v)
            v = jnp.exp(v)
            eb[e, :] = v
            eidx = jnp.full((LANES,), e, jnp.int32)
            for j in range(4):
                widx = jnp.full((LANES,), 4 * c + j, jnp.int32)
                wv = plsc.load_gather(eb, [eidx, widx])
                for t in range(2):
                    col = 32 * j + LANES * t
                    hb[e, pl.ds(col, LANES)] = hb[e, pl.ds(col, LANES)] * wv

        pltpu.sync_copy(hb, out_sh.at[idst], add=True)

        @pl.when(c == 0)
        def _wea():
            pltpu.sync_copy(eb, ea_hbm.at[pl.ds(base, CHUNK)])

    plsc.subcore_barrier()

    @pl.loop(0, NQ)
    def _drain(q):
        cid = s + NTILE * q

        @pl.when(cid < NROUND)
        def _():
            pltpu.sync_copy(riota_hbm.at[pl.ds(cid * NR, NR)], ridx)
            pltpu.sync_copy(out_sh.at[ridx], zbuf)
            pltpu.sync_copy(zbuf, outp_hbm.at[pl.ds(c * NPAD + cid * NR, NR)])


def _s2_body(dst_hbm, ea_hbm, riota_hbm, asum_hbm, asum_sh, idst, eb, wb,
             zbuf, ridx):
    c = lax.axis_index("c")
    s = lax.axis_index("s")

    @pl.loop(0, NR)
    def _zb(i):
        for j in range(HALF // LANES):
            zbuf[i, pl.ds(LANES * j, LANES)] = jnp.zeros((LANES,), jnp.float32)

    @pl.loop(0, CHUNK)
    def _zw(i):
        for j in range(HALF // LANES):
            wb[i, pl.ds(LANES * j, LANES)] = jnp.zeros((LANES,), jnp.float32)

    @pl.loop(0, NQ)
    def _init(q):
        cid = s + NTILE * q

        @pl.when(cid < NROUND)
        def _():
            pltpu.sync_copy(riota_hbm.at[pl.ds(cid * NR, NR)], ridx)
            pltpu.sync_copy(zbuf, asum_sh.at[ridx])

    plsc.subcore_barrier()

    @pl.loop(0, NCHUNK2)
    def _chunk(k):
        base = (c * NTILE + s) * EPT2 + k * CHUNK
        pltpu.sync_copy(dst_hbm.at[pl.ds(base, CHUNK)], idst)
        pltpu.sync_copy(ea_hbm.at[pl.ds(base, CHUNK)], eb)

        @pl.loop(0, CHUNK)
        def _edge(e):
            wb[e, 0:LANES] = eb[e, :]

        pltpu.sync_copy(wb, asum_sh.at[idst], add=True)

    plsc.subcore_barrier()

    @pl.loop(0, NQ)
    def _drain(q):
        cid = s + NTILE * q

        @pl.when(cid < NROUND)
        def _():
            pltpu.sync_copy(riota_hbm.at[pl.ds(cid * NR, NR)], ridx)
            pltpu.sync_copy(asum_sh.at[ridx], zbuf)
            pltpu.sync_copy(zbuf, asum_hbm.at[pl.ds(c * NPAD + cid * NR, NR)])


def _head_body(outp_ref, asum_ref, r_ref, bias_ref, w1_ref, b1_ref, w2_ref,
               b2_ref, o_ref):
    asum = asum_ref[0:N, 0:LANES] + asum_ref[NPAD:NPAD + N, 0:LANES]
    inv = 1.0 / (asum + 1e-16)
    inv_exp = jnp.dot(inv, r_ref[...], preferred_element_type=jnp.float32)
    o = jnp.concatenate([outp_ref[0:N, :], outp_ref[NPAD:NPAD + N, :]], axis=1)
    o = o * inv_exp + bias_ref[...]
    o = jnp.where(o > 0.0, o, jnp.exp(o) - 1.0)
    pooled = jnp.sum(o, axis=0, keepdims=True) * (1.0 / N)
    z = jnp.dot(pooled, w1_ref[...], preferred_element_type=jnp.float32)
    z = jnp.maximum(z + b1_ref[...], 0.0)
    o_ref[...] = jnp.dot(z, w2_ref[...],
                         preferred_element_type=jnp.float32) + b2_ref[...]


def kernel(x, edge_index, W_gat, att_src, att_dst, gat_bias, W1, b1, W2, b2):
    ei = edge_index.astype(jnp.int32)
    loops = jnp.arange(N, dtype=jnp.int32)
    padi = jnp.full((EPAD - ETOT,), N, jnp.int32)
    src = jnp.concatenate([ei[0], loops, padi])
    dst = jnp.concatenate([ei[1], loops, padi])
    riota = jnp.arange(NPAD, dtype=jnp.int32)

    asrc = att_src.reshape(1, HEADS * C).astype(jnp.float32)
    adst = att_dst.reshape(1, HEADS * C).astype(jnp.float32)
    g = jnp.arange(HEADS * C)[:, None] // C
    k16 = jnp.arange(LANES)[None, :] % HEADS
    m = (g == k16).astype(jnp.float32)
    r = 0.5 * m.T

    h3, ab = pl.pallas_call(
        _prep_body,
        out_shape=[
            jax.ShapeDtypeStruct((NSC * NPAD, HALF), jnp.float32),
            jax.ShapeDtypeStruct((NPAD, HALF), jnp.float32),
        ],
    )(x, W_gat, asrc, adst, m)

    mesh = plsc.VectorSubcoreMesh(core_axis_name="c", subcore_axis_name="s")
    cp = pltpu.CompilerParams()
    if "needs_layout_passes" in pltpu.CompilerParams.__dataclass_fields__:
        cp = dataclasses.replace(cp, needs_layout_passes=False)

    s1 = pl.kernel(
        _s1_body,
        out_type=[
            jax.ShapeDtypeStruct((NSC * NPAD, HALF), jnp.float32),
            jax.ShapeDtypeStruct((EPAD, LANES), jnp.float32),
        ],
        mesh=mesh,
        compiler_params=cp,
        scratch_types=[
            pltpu.VMEM_SHARED((NPAD, HALF), jnp.float32),
            pltpu.VMEM((CHUNK,), jnp.int32),
            pltpu.VMEM((CHUNK,), jnp.int32),
            pltpu.VMEM((CHUNK, HALF), jnp.float32),
            pltpu.VMEM((CHUNK, HALF), jnp.float32),
            pltpu.VMEM((CHUNK, LANES), jnp.float32),
            pltpu.VMEM((CHUNK, HALF), jnp.float32),
            pltpu.VMEM((NR, HALF), jnp.float32),
            pltpu.VMEM((NR,), jnp.int32),
        ],
    )
    outp, ea = s1(src, dst, ab, h3, riota)

    s2 = pl.kernel(
        _s2_body,
        out_type=jax.ShapeDtypeStruct((NSC * NPAD, HALF), jnp.float32),
        mesh=mesh,
        compiler_params=cp,
        scratch_types=[
            pltpu.VMEM_SHARED((NPAD, HALF), jnp.float32),
            pltpu.VMEM((CHUNK,), jnp.int32),
            pltpu.VMEM((CHUNK, LANES), jnp.float32),
            pltpu.VMEM((CHUNK, HALF), jnp.float32),
            pltpu.VMEM((NR, HALF), jnp.float32),
            pltpu.VMEM((NR,), jnp.int32),
        ],
    )
    asum = s2(dst, ea, riota)

    logits = pl.pallas_call(
        _head_body,
        out_shape=jax.ShapeDtypeStruct((1, NUM_CLASSES), jnp.float32),
    )(outp, asum, r, gat_bias.astype(jnp.float32), W1, b1, W2, b2)
    return logits
